# baseline (speedup 1.0000x reference)
import functools

import jax
import jax.numpy as jnp
from jax.experimental import pallas as pl
from jax.experimental.pallas import tpu as pltpu

_CP = pltpu.CompilerParams(
    dimension_semantics=("arbitrary",),
    vmem_limit_bytes=60 * 1024 * 1024,
)


def _taps(w):
    co, ci, kh, kw = w.shape
    t = jnp.transpose(w, (2, 3, 0, 1)).reshape(kh * kw, co, ci)
    return t.astype(jnp.bfloat16)


def _prep(x, pad, w_):
    xp = jnp.pad(x, ((0, 0), (pad, pad)))
    s = jax.lax.broadcasted_iota(jnp.int32, (1, xp.shape[1]), 1)
    wp = (s - pad) & (w_ - 1)
    zero = jnp.zeros_like(xp)
    xm = jnp.where(wp != w_ - 1, xp, zero)
    xr = jnp.where(wp != 0, xp, zero)
    return (xm, xp, xr)


def _conv9(wt_ref, srcs, acc, tm, dpad, w_):
    for t in range(9):
        dh, dw = t // 3 - 1, t % 3 - 1
        d = dpad + dh * w_ + dw
        sl = jax.lax.slice_in_dim(srcs[dw + 1], d, d + tm, axis=1)
        acc = acc + jnp.dot(wt_ref[t], sl, preferred_element_type=jnp.float32)
    return acc


def _fused_kernel(wh2h_ref, wh2l_ref, wll_ref, pool_ref, up_ref,
                  gbh_ref, gbl_ref, xh_ref, xl_ref,
                  oh_ref, ol_ref,
                  yh_s, yl_s, pa_h, pa_l,
                  *, h, w, hl, wl, nb, nb2, na, mh, ml, eps):
    hw, hwl = h * w, hl * wl
    col = ol_ref.shape[1]
    coh = oh_ref.shape[1]
    i = pl.program_id(0)

    @pl.when(i < na)
    def _phase_a():
        pl_pad = wl + 1
        ph_pad = w + 1

        sum_h = jnp.zeros((coh, 2), jnp.float32)
        sum_l = jnp.zeros((col, 2), jnp.float32)
        for j in range(nb):
            xh = xh_ref[j].astype(jnp.bfloat16)
            xl = xl_ref[j].astype(jnp.bfloat16)
            pooled = jnp.dot(xh, pool_ref[...],
                             preferred_element_type=jnp.float32)
            pooled = pooled.astype(jnp.bfloat16)

            xlp = _prep(xl, pl_pad, wl)
            plp = _prep(pooled, pl_pad, wl)
            xhp = _prep(xh, ph_pad, w)

            acc_ll = jnp.zeros((col + coh, hwl), jnp.float32)
            acc_ll = _conv9(wll_ref, xlp, acc_ll, hwl, pl_pad, wl)
            acc_lo = _conv9(wh2l_ref, plp, acc_ll[:col], hwl, pl_pad, wl)
            acc_l2h = acc_ll[col:]

            acc_hi = jnp.dot(acc_l2h.astype(jnp.bfloat16), up_ref[...],
                             preferred_element_type=jnp.float32)
            acc_hi = _conv9(wh2h_ref, xhp, acc_hi, hw, ph_pad, w)

            yh_s[pl.ds(i * nb + j, 1)] = acc_hi.astype(yh_s.dtype)[None]
            yl_s[pl.ds(i * nb + j, 1)] = acc_lo.astype(yl_s.dtype)[None]
            sum_h = sum_h + jnp.concatenate(
                [jnp.sum(acc_hi, axis=1, keepdims=True),
                 jnp.sum(acc_hi * acc_hi, axis=1, keepdims=True)], axis=1)
            sum_l = sum_l + jnp.concatenate(
                [jnp.sum(acc_lo, axis=1, keepdims=True),
                 jnp.sum(acc_lo * acc_lo, axis=1, keepdims=True)], axis=1)

        @pl.when(i == 0)
        def _init():
            pa_h[...] = sum_h
            pa_l[...] = sum_l

        @pl.when(i > 0)
        def _accum():
            pa_h[...] = pa_h[...] + sum_h
            pa_l[...] = pa_l[...] + sum_l

    @pl.when(i >= na)
    def _phase_b():
        k = i - na

        def fold(pa, gb, m):
            mean = pa[:, 0:1] / m
            var = pa[:, 1:2] / m - mean * mean
            scale = gb[:, 0:1] * jax.lax.rsqrt(var + eps)
            shift = gb[:, 1:2] - mean * scale
            return scale, shift

        sc_h, sh_h = fold(pa_h[...], gbh_ref[...], mh)
        sc_l, sh_l = fold(pa_l[...], gbl_ref[...], ml)
        yh = yh_s[pl.ds(k * nb2, nb2)].astype(jnp.float32)
        yl = yl_s[pl.ds(k * nb2, nb2)].astype(jnp.float32)
        oh_ref[...] = yh * sc_h + sh_h
        ol_ref[...] = yl * sc_l + sh_l


def kernel(w_h2h, w_h2l, w_l2h, w_l2l, gamma_h, beta_h, gamma_l, beta_l,
           x_h, x_l, eps=1e-5):
    b, cih, h, w = x_h.shape
    _, cil, hl, wl = x_l.shape
    coh = w_h2h.shape[0]
    col = w_l2l.shape[0]
    assert h & (h - 1) == 0 and w & (w - 1) == 0, "spatial dims must be pow2"
    hw, hwl = h * w, hl * wl
    mh, ml = b * hw, b * hwl

    q = jnp.arange(hw)
    p_of_q = (q // (2 * w)) * wl + (q % w) // 2
    sel = p_of_q[:, None] == jnp.arange(hwl)[None, :]
    pool_mat = jnp.where(sel, 0.25, 0.0).astype(jnp.bfloat16)
    up_mat = jnp.where(sel, 1.0, 0.0).astype(jnp.bfloat16).T

    nb = 4 if b % 4 == 0 else 1
    nb2 = 8 if b % 8 == 0 else 1
    na, nbb = b // nb, b // nb2
    gb_h = jnp.stack([gamma_h, beta_h], axis=1).astype(jnp.float32)
    gb_l = jnp.stack([gamma_l, beta_l], axis=1).astype(jnp.float32)

    kfn = functools.partial(_fused_kernel, h=h, w=w, hl=hl, wl=wl,
                            nb=nb, nb2=nb2, na=na, mh=mh, ml=ml, eps=eps)
    out_h, out_l = pl.pallas_call(
        kfn,
        out_shape=(jax.ShapeDtypeStruct((b, coh, hw), jnp.float32),
                   jax.ShapeDtypeStruct((b, col, hwl), jnp.float32)),
        grid=(na + nbb,),
        in_specs=[
            pl.BlockSpec((9, coh, cih), lambda i: (0, 0, 0)),
            pl.BlockSpec((9, col, cih), lambda i: (0, 0, 0)),
            pl.BlockSpec((9, col + coh, cil), lambda i: (0, 0, 0)),
            pl.BlockSpec((hw, hwl), lambda i: (0, 0)),
            pl.BlockSpec((hwl, hw), lambda i: (0, 0)),
            pl.BlockSpec((coh, 2), lambda i: (0, 0)),
            pl.BlockSpec((col, 2), lambda i: (0, 0)),
            pl.BlockSpec((nb, cih, hw),
                         lambda i: (jnp.minimum(i, na - 1), 0, 0)),
            pl.BlockSpec((nb, cil, hwl),
                         lambda i: (jnp.minimum(i, na - 1), 0, 0)),
        ],
        out_specs=(pl.BlockSpec((nb2, coh, hw),
                                lambda i: (jnp.maximum(i - na, 0), 0, 0)),
                   pl.BlockSpec((nb2, col, hwl),
                                lambda i: (jnp.maximum(i - na, 0), 0, 0))),
        scratch_shapes=[
            pltpu.VMEM((b, coh, hw), jnp.bfloat16),
            pltpu.VMEM((b, col, hwl), jnp.bfloat16),
            pltpu.VMEM((coh, 2), jnp.float32),
            pltpu.VMEM((col, 2), jnp.float32),
        ],
        compiler_params=_CP,
    )(_taps(w_h2h), _taps(w_h2l),
      jnp.concatenate([_taps(w_l2l), _taps(w_l2h)], axis=1),
      pool_mat, up_mat, gb_h, gb_l,
      x_h.reshape(b, cih, hw), x_l.reshape(b, cil, hwl))

    return out_h.reshape(b, coh, h, w), out_l.reshape(b, col, hl, wl)


# speedup vs baseline: 1.0068x; 1.0068x over previous
import functools

import jax
import jax.numpy as jnp
from jax.experimental import pallas as pl
from jax.experimental.pallas import tpu as pltpu

_CP = pltpu.CompilerParams(
    dimension_semantics=("arbitrary",),
    vmem_limit_bytes=60 * 1024 * 1024,
)


def _taps(w):
    co, ci, kh, kw = w.shape
    t = jnp.transpose(w, (2, 3, 0, 1)).reshape(kh * kw, co, ci)
    return t.astype(jnp.bfloat16)


def _prep(x, pad, w_):
    xp = jnp.pad(x, ((0, 0), (pad, pad)))
    s = jax.lax.broadcasted_iota(jnp.int32, (1, xp.shape[1]), 1)
    wp = (s - pad) & (w_ - 1)
    zero = jnp.zeros_like(xp)
    xm = jnp.where(wp != w_ - 1, xp, zero)
    xr = jnp.where(wp != 0, xp, zero)
    return (xm, xp, xr)


def _conv9(wt_ref, srcs, acc, tm, dpad, w_):
    for t in range(9):
        dh, dw = t // 3 - 1, t % 3 - 1
        d = dpad + dh * w_ + dw
        sl = jax.lax.slice_in_dim(srcs[dw + 1], d, d + tm, axis=1)
        acc = acc + jnp.dot(wt_ref[t], sl, preferred_element_type=jnp.float32)
    return acc


def _fused_kernel(wh2h_ref, wh2l_ref, wll_ref, pool_ref, up_ref,
                  gbh_ref, gbl_ref, xh_ref, xl_ref,
                  oh_ref, ol_ref,
                  yh_s, yl_s, pa_h, pa_l,
                  *, h, w, hl, wl, nb, nb2, na, mh, ml, eps):
    hw, hwl = h * w, hl * wl
    col = ol_ref.shape[1]
    coh = oh_ref.shape[1]
    i = pl.program_id(0)

    @pl.when(i < na)
    def _phase_a():
        pl_pad = wl + 1
        ph_pad = w + 1

        sum_h = jnp.zeros((coh, 2), jnp.float32)
        sum_l = jnp.zeros((col, 2), jnp.float32)
        for j in range(nb):
            xh = xh_ref[j].astype(jnp.bfloat16)
            xl = xl_ref[j].astype(jnp.bfloat16)
            pooled = jnp.dot(xh, pool_ref[...],
                             preferred_element_type=jnp.float32)
            pooled = pooled.astype(jnp.bfloat16)

            xlp = _prep(xl, pl_pad, wl)
            plp = _prep(pooled, pl_pad, wl)
            xhp = _prep(xh, ph_pad, w)

            acc_ll = jnp.zeros((col + coh, hwl), jnp.float32)
            acc_ll = _conv9(wll_ref, xlp, acc_ll, hwl, pl_pad, wl)
            acc_lo = _conv9(wh2l_ref, plp, acc_ll[:col], hwl, pl_pad, wl)
            acc_l2h = acc_ll[col:]

            acc_hi = jnp.dot(acc_l2h.astype(jnp.bfloat16), up_ref[...],
                             preferred_element_type=jnp.float32)
            acc_hi = _conv9(wh2h_ref, xhp, acc_hi, hw, ph_pad, w)

            yh_s[pl.ds(i * nb + j, 1)] = acc_hi.astype(yh_s.dtype)[None]
            yl_s[pl.ds(i * nb + j, 1)] = acc_lo.astype(yl_s.dtype)[None]
            sum_h = sum_h + jnp.concatenate(
                [jnp.sum(acc_hi, axis=1, keepdims=True),
                 jnp.sum(acc_hi * acc_hi, axis=1, keepdims=True)], axis=1)
            sum_l = sum_l + jnp.concatenate(
                [jnp.sum(acc_lo, axis=1, keepdims=True),
                 jnp.sum(acc_lo * acc_lo, axis=1, keepdims=True)], axis=1)

        @pl.when(i == 0)
        def _init():
            pa_h[...] = sum_h
            pa_l[...] = sum_l

        @pl.when(i > 0)
        def _accum():
            pa_h[...] = pa_h[...] + sum_h
            pa_l[...] = pa_l[...] + sum_l

    @pl.when(i >= na)
    def _phase_b():
        k = i - na

        def fold(pa, gb, m):
            mean = pa[:, 0:1] / m
            var = pa[:, 1:2] / m - mean * mean
            scale = gb[:, 0:1] * jax.lax.rsqrt(var + eps)
            shift = gb[:, 1:2] - mean * scale
            return scale, shift

        sc_h, sh_h = fold(pa_h[...], gbh_ref[...], mh)
        sc_l, sh_l = fold(pa_l[...], gbl_ref[...], ml)
        yh = yh_s[pl.ds(k * nb2, nb2)].astype(jnp.float32)
        yl = yl_s[pl.ds(k * nb2, nb2)].astype(jnp.float32)
        oh_ref[...] = yh * sc_h + sh_h
        ol_ref[...] = yl * sc_l + sh_l


def kernel(w_h2h, w_h2l, w_l2h, w_l2l, gamma_h, beta_h, gamma_l, beta_l,
           x_h, x_l, eps=1e-5):
    b, cih, h, w = x_h.shape
    _, cil, hl, wl = x_l.shape
    coh = w_h2h.shape[0]
    col = w_l2l.shape[0]
    assert h & (h - 1) == 0 and w & (w - 1) == 0, "spatial dims must be pow2"
    hw, hwl = h * w, hl * wl
    mh, ml = b * hw, b * hwl

    q = jnp.arange(hw)
    p_of_q = (q // (2 * w)) * wl + (q % w) // 2
    sel = p_of_q[:, None] == jnp.arange(hwl)[None, :]
    pool_mat = jnp.where(sel, 0.25, 0.0).astype(jnp.bfloat16)
    up_mat = jnp.where(sel, 1.0, 0.0).astype(jnp.bfloat16).T

    nb = 8 if b % 8 == 0 else 1
    nb2 = 8 if b % 8 == 0 else 1
    na, nbb = b // nb, b // nb2
    gb_h = jnp.stack([gamma_h, beta_h], axis=1).astype(jnp.float32)
    gb_l = jnp.stack([gamma_l, beta_l], axis=1).astype(jnp.float32)

    kfn = functools.partial(_fused_kernel, h=h, w=w, hl=hl, wl=wl,
                            nb=nb, nb2=nb2, na=na, mh=mh, ml=ml, eps=eps)
    out_h, out_l = pl.pallas_call(
        kfn,
        out_shape=(jax.ShapeDtypeStruct((b, coh, hw), jnp.float32),
                   jax.ShapeDtypeStruct((b, col, hwl), jnp.float32)),
        grid=(na + nbb,),
        in_specs=[
            pl.BlockSpec((9, coh, cih), lambda i: (0, 0, 0)),
            pl.BlockSpec((9, col, cih), lambda i: (0, 0, 0)),
            pl.BlockSpec((9, col + coh, cil), lambda i: (0, 0, 0)),
            pl.BlockSpec((hw, hwl), lambda i: (0, 0)),
            pl.BlockSpec((hwl, hw), lambda i: (0, 0)),
            pl.BlockSpec((coh, 2), lambda i: (0, 0)),
            pl.BlockSpec((col, 2), lambda i: (0, 0)),
            pl.BlockSpec((nb, cih, hw),
                         lambda i: (jnp.minimum(i, na - 1), 0, 0)),
            pl.BlockSpec((nb, cil, hwl),
                         lambda i: (jnp.minimum(i, na - 1), 0, 0)),
        ],
        out_specs=(pl.BlockSpec((nb2, coh, hw),
                                lambda i: (jnp.maximum(i - na, 0), 0, 0)),
                   pl.BlockSpec((nb2, col, hwl),
                                lambda i: (jnp.maximum(i - na, 0), 0, 0))),
        scratch_shapes=[
            pltpu.VMEM((b, coh, hw), jnp.bfloat16),
            pltpu.VMEM((b, col, hwl), jnp.bfloat16),
            pltpu.VMEM((coh, 2), jnp.float32),
            pltpu.VMEM((col, 2), jnp.float32),
        ],
        compiler_params=_CP,
    )(_taps(w_h2h), _taps(w_h2l),
      jnp.concatenate([_taps(w_l2l), _taps(w_l2h)], axis=1),
      pool_mat, up_mat, gb_h, gb_l,
      x_h.reshape(b, cih, hw), x_l.reshape(b, cil, hwl))

    return out_h.reshape(b, coh, h, w), out_l.reshape(b, col, hl, wl)
